# baseline (speedup 1.0000x reference)
import jax
import jax.numpy as jnp
from jax import lax
from jax.experimental import pallas as pl
from jax.experimental.pallas import tpu as pltpu
from jax.experimental.pallas import tpu_sc as plsc
from jax.experimental.layout import Format, Layout, with_layout_constraint

B = 1024
CTX = 10
D = 64
V = 100000


_NW = 32
_BPW = B // _NW
_IPW = _BPW * CTX
_CHUNK = 80
_NCHUNK = _IPW // _CHUNK


def _sc_body(x_hbm, emb_hbm, out_hbm, idx_v, rows_v, out_v, sem):
    wid = lax.axis_index("s") * 2 + lax.axis_index("c")
    base = wid * _IPW
    pltpu.sync_copy(x_hbm.at[pl.ds(base, _IPW)], idx_v)
    copies = []
    for k in range(_NCHUNK):
        copies.append(
            pltpu.async_copy(
                emb_hbm.at[idx_v.at[pl.ds(k * _CHUNK, _CHUNK)]],
                rows_v.at[pl.ds(k * _CHUNK, _CHUNK)],
                sem,
            )
        )
    for c in copies:
        c.wait()

    def row(r, carry):
        for j in range(D // 16):
            sl = pl.ds(j * 16, 16)
            acc = rows_v[r * CTX, sl]
            for c in range(1, CTX):
                acc = acc + rows_v[r * CTX + c, sl]
            out_v[r, sl] = acc
        return carry

    lax.fori_loop(0, _BPW, row, 0)
    pltpu.sync_copy(out_v, out_hbm.at[pl.ds(wid * _BPW, _BPW)])


def _sc_gather_sum(x_flat, emb):
    mesh = plsc.VectorSubcoreMesh(core_axis_name="c", subcore_axis_name="s")
    k = pl.kernel(
        _sc_body,
        mesh=mesh,
        out_type=jax.ShapeDtypeStruct((B, D), jnp.float32),
        scratch_types=[
            pltpu.VMEM((_IPW,), jnp.int32),
            pltpu.VMEM((_IPW, D), jnp.float32),
            pltpu.VMEM((_BPW, D), jnp.float32),
            pltpu.SemaphoreType.DMA,
        ],
        compiler_params=pltpu.CompilerParams(use_tc_tiling_on_sc=False),
    )
    return k(x_flat, emb)


_VB = 2048
_NV = -(-V // _VB)


def _tc_body(s_ref, w_ref, b_ref, out_ref, m_ref, l_ref, lse_ref):
    p = pl.program_id(0)
    j = pl.program_id(1)
    nv = pl.num_programs(1)
    s = s_ref[...]
    logits = (
        lax.dot_general(
            s, w_ref[...],
            dimension_numbers=(((1,), (1,)), ((), ())),
            preferred_element_type=jnp.float32,
        )
        + b_ref[...]
    )
    col = j * _VB + lax.broadcasted_iota(jnp.int32, (1, _VB), 1)
    valid = col < V

    @pl.when(p == 0)
    def _():
        @pl.when(j == 0)
        def _():
            m_ref[...] = jnp.full((B, 1), -jnp.inf, jnp.float32)
            l_ref[...] = jnp.zeros((B, 1), jnp.float32)

        lm = jnp.where(valid, logits, -jnp.inf)
        m_old = m_ref[...]
        m_new = jnp.maximum(m_old, jnp.max(lm, axis=1, keepdims=True))
        l_ref[...] = l_ref[...] * jnp.exp(m_old - m_new) + jnp.sum(
            jnp.where(valid, jnp.exp(lm - m_new), 0.0), axis=1, keepdims=True
        )
        m_ref[...] = m_new

        @pl.when(j == nv - 1)
        def _():
            lse_ref[...] = m_new + jnp.log(l_ref[...])

    @pl.when(p == 1)
    def _():
        out_ref[...] = logits - lse_ref[...]


def _tc_logsoftmax(s, W, b2):
    return pl.pallas_call(
        _tc_body,
        grid=(2, _NV),
        in_specs=[
            pl.BlockSpec((B, D), lambda p, j: (0, 0)),
            pl.BlockSpec((_VB, D), lambda p, j: (j, 0)),
            pl.BlockSpec((1, _VB), lambda p, j: (0, j)),
        ],
        out_specs=pl.BlockSpec((B, _VB), lambda p, j: (0, j * p)),
        out_shape=jax.ShapeDtypeStruct((B, V), jnp.float32),
        scratch_shapes=[
            pltpu.VMEM((B, 1), jnp.float32),
            pltpu.VMEM((B, 1), jnp.float32),
            pltpu.VMEM((B, 1), jnp.float32),
        ],
    )(s, W, b2)


def _impl(x, emb, W, b):
    s = _sc_gather_sum(x.reshape(-1).astype(jnp.int32), emb)
    return _tc_logsoftmax(s, W, b.reshape(1, V))


_jit_cache = {}


def kernel(x, emb, W, b):
    if isinstance(x, jax.core.Tracer) or not hasattr(x, "devices"):
        return _impl(x, emb, W, b)
    dev = next(iter(x.devices()))
    if dev not in _jit_cache:
        fmt = Format(Layout((1, 0)), jax.sharding.SingleDeviceSharding(dev))
        _jit_cache[dev] = jax.jit(_impl, out_shardings=fmt)
    return _jit_cache[dev](x, emb, W, b)


# speedup vs baseline: 1.9263x; 1.9263x over previous
import jax
import jax.numpy as jnp
from jax import lax
from jax.experimental import pallas as pl
from jax.experimental.pallas import tpu as pltpu
from jax.experimental.pallas import tpu_sc as plsc

B = 1024
CTX = 10
D = 64
V = 100000


_NW = 32
_BPW = B // _NW
_IPW = _BPW * CTX
_CHUNK = 80
_NCHUNK = _IPW // _CHUNK


def _sc_body(x_hbm, emb_hbm, out_hbm, idx_v, rows_v, out_v, sem):
    wid = lax.axis_index("s") * 2 + lax.axis_index("c")
    base = wid * _IPW
    pltpu.sync_copy(x_hbm.at[pl.ds(base, _IPW)], idx_v)
    copies = []
    for k in range(_NCHUNK):
        copies.append(
            pltpu.async_copy(
                emb_hbm.at[idx_v.at[pl.ds(k * _CHUNK, _CHUNK)]],
                rows_v.at[pl.ds(k * _CHUNK, _CHUNK)],
                sem,
            )
        )
    for c in copies:
        c.wait()

    def row(r, carry):
        for j in range(D // 16):
            sl = pl.ds(j * 16, 16)
            acc = rows_v[r * CTX, sl]
            for c in range(1, CTX):
                acc = acc + rows_v[r * CTX + c, sl]
            out_v[r, sl] = acc
        return carry

    lax.fori_loop(0, _BPW, row, 0)
    pltpu.sync_copy(out_v, out_hbm.at[pl.ds(wid * _BPW, _BPW)])


def _sc_gather_sum(x_flat, emb):
    mesh = plsc.VectorSubcoreMesh(core_axis_name="c", subcore_axis_name="s")
    k = pl.kernel(
        _sc_body,
        mesh=mesh,
        out_type=jax.ShapeDtypeStruct((B, D), jnp.float32),
        scratch_types=[
            pltpu.VMEM((_IPW,), jnp.int32),
            pltpu.VMEM((_IPW, D), jnp.float32),
            pltpu.VMEM((_BPW, D), jnp.float32),
            pltpu.SemaphoreType.DMA,
        ],
        compiler_params=pltpu.CompilerParams(use_tc_tiling_on_sc=False),
    )
    return k(x_flat, emb)


_VB = 2048
_NV = -(-V // _VB)


def _tc_body(s_ref, wt_ref, b_ref, out_ref, m_ref, l_ref, lse_ref):
    p = pl.program_id(0)
    j = pl.program_id(1)
    nv = pl.num_programs(1)
    s_aug = jnp.concatenate(
        [s_ref[...], jnp.ones((B, 1), jnp.float32)], axis=1
    )
    wt_aug = jnp.concatenate([wt_ref[...], b_ref[...]], axis=0)
    t = lax.dot_general(
        wt_aug, s_aug,
        dimension_numbers=(((0,), (1,)), ((), ())),
        preferred_element_type=jnp.float32,
    )
    row = j * _VB + lax.broadcasted_iota(jnp.int32, (_VB, 1), 0)
    valid = row < V

    @pl.when(p == 0)
    def _():
        @pl.when(j == 0)
        def _():
            m_ref[...] = jnp.full((1, B), -jnp.inf, jnp.float32)
            l_ref[...] = jnp.zeros((1, B), jnp.float32)

        lm = jnp.where(valid, t, -jnp.inf)
        m_old = m_ref[...]
        m_new = jnp.maximum(m_old, jnp.max(lm, axis=0, keepdims=True))
        l_ref[...] = l_ref[...] * jnp.exp(m_old - m_new) + jnp.sum(
            jnp.exp(lm - m_new), axis=0, keepdims=True
        )
        m_ref[...] = m_new

        @pl.when(j == nv - 1)
        def _():
            lse_ref[...] = m_new + jnp.log(l_ref[...])

    @pl.when(p == 1)
    def _():
        out_ref[...] = t - lse_ref[...]


def _tc_logsoftmax(s, Wt, b2):
    out_t = pl.pallas_call(
        _tc_body,
        grid=(2, _NV),
        in_specs=[
            pl.BlockSpec((B, D), lambda p, j: (0, 0)),
            pl.BlockSpec((D, _VB), lambda p, j: (0, j)),
            pl.BlockSpec((1, _VB), lambda p, j: (0, j)),
        ],
        out_specs=pl.BlockSpec((_VB, B), lambda p, j: (j * p, 0)),
        out_shape=jax.ShapeDtypeStruct((V, B), jnp.float32),
        scratch_shapes=[
            pltpu.VMEM((1, B), jnp.float32),
            pltpu.VMEM((1, B), jnp.float32),
            pltpu.VMEM((1, B), jnp.float32),
        ],
    )(s, Wt, b2)
    return out_t.T


@jax.jit
def kernel(x, emb, W, b):
    s = _sc_gather_sum(x.reshape(-1).astype(jnp.int32), emb)
    return _tc_logsoftmax(s, W.T, b.reshape(1, V))


# speedup vs baseline: 2.0950x; 1.0876x over previous
import jax
import jax.numpy as jnp
from jax import lax
from jax.experimental import pallas as pl
from jax.experimental.pallas import tpu as pltpu
from jax.experimental.pallas import tpu_sc as plsc

B = 1024
CTX = 10
D = 64
V = 100000


_NW = 32
_BPW = B // _NW
_IPW = _BPW * CTX
_CHUNK = 80
_NCHUNK = _IPW // _CHUNK


def _sc_body(x_hbm, emb_hbm, out_hbm, idx_v, rows_v, out_v, sem):
    wid = lax.axis_index("s") * 2 + lax.axis_index("c")
    base = wid * _IPW
    pltpu.sync_copy(x_hbm.at[pl.ds(base, _IPW)], idx_v)
    copies = []
    for k in range(_NCHUNK):
        copies.append(
            pltpu.async_copy(
                emb_hbm.at[idx_v.at[pl.ds(k * _CHUNK, _CHUNK)]],
                rows_v.at[pl.ds(k * _CHUNK, _CHUNK)],
                sem,
            )
        )
    for c in copies:
        c.wait()

    def row(r, carry):
        for j in range(D // 16):
            sl = pl.ds(j * 16, 16)
            acc = rows_v[r * CTX, sl]
            for c in range(1, CTX):
                acc = acc + rows_v[r * CTX + c, sl]
            out_v[r, sl] = acc
        return carry

    lax.fori_loop(0, _BPW, row, 0)
    pltpu.sync_copy(out_v, out_hbm.at[pl.ds(wid * _BPW, _BPW)])


def _sc_gather_sum(x_flat, emb):
    mesh = plsc.VectorSubcoreMesh(core_axis_name="c", subcore_axis_name="s")
    k = pl.kernel(
        _sc_body,
        mesh=mesh,
        out_type=jax.ShapeDtypeStruct((B, D), jnp.float32),
        scratch_types=[
            pltpu.VMEM((_IPW,), jnp.int32),
            pltpu.VMEM((_IPW, D), jnp.float32),
            pltpu.VMEM((_BPW, D), jnp.float32),
            pltpu.SemaphoreType.DMA,
        ],
        compiler_params=pltpu.CompilerParams(use_tc_tiling_on_sc=False),
    )
    return k(x_flat, emb)


_VB = 2048
_NV = -(-V // _VB)
_VPAD = _NV * _VB
_K = D + 2
_LOG2E = 1.4426950408889634
_LN2 = 0.6931471805599453


def _tc_body(s_ref, wt_ref, b_ref, out_ref, s2_ref, m_ref, l_ref):
    p = pl.program_id(0)
    j = pl.program_id(1)
    nv = pl.num_programs(1)

    @pl.when((p == 0) & (j == 0))
    def _():
        s2_ref[:, :D] = s_ref[...]
        s2_ref[:, D:D + 1] = jnp.ones((B, 1), jnp.float32)
        s2_ref[:, D + 1:] = jnp.zeros((B, 1), jnp.float32)
        m_ref[...] = jnp.full((1, B), -jnp.inf, jnp.float32)
        l_ref[...] = jnp.zeros((1, B), jnp.float32)

    scale = jnp.where(p == 0, jnp.float32(_LOG2E), jnp.float32(1.0))
    wt2 = (
        jnp.concatenate(
            [wt_ref[...], b_ref[...], jnp.full((1, _VB), -1.0, jnp.float32)],
            axis=0,
        )
        * scale
    )
    t = lax.dot_general(
        wt2, s2_ref[...],
        dimension_numbers=(((0,), (1,)), ((), ())),
        preferred_element_type=jnp.float32,
    )

    @pl.when(p == 0)
    def _():
        m_old = m_ref[...]
        m_new = jnp.maximum(m_old, jnp.max(t, axis=0, keepdims=True))
        l_ref[...] = l_ref[...] * jnp.exp2(m_old - m_new) + jnp.sum(
            jnp.exp2(t - m_new), axis=0, keepdims=True
        )
        m_ref[...] = m_new

        @pl.when(j == nv - 1)
        def _():
            lse = m_new * _LN2 + jnp.log(l_ref[...])
            s2_ref[:, D + 1:] = jnp.transpose(lse)

    @pl.when(p == 1)
    def _():
        out_ref[...] = t


def _tc_logsoftmax(s, Wt_pad, b2_pad):
    out_t = pl.pallas_call(
        _tc_body,
        grid=(2, _NV),
        in_specs=[
            pl.BlockSpec((B, D), lambda p, j: (0, 0)),
            pl.BlockSpec((D, _VB), lambda p, j: (0, j)),
            pl.BlockSpec((1, _VB), lambda p, j: (0, j)),
        ],
        out_specs=pl.BlockSpec((_VB, B), lambda p, j: (j * p, 0)),
        out_shape=jax.ShapeDtypeStruct((V, B), jnp.float32),
        scratch_shapes=[
            pltpu.VMEM((B, _K), jnp.float32),
            pltpu.VMEM((1, B), jnp.float32),
            pltpu.VMEM((1, B), jnp.float32),
        ],
    )(s, Wt_pad, b2_pad)
    return out_t.T


@jax.jit
def kernel(x, emb, W, b):
    s = _sc_gather_sum(x.reshape(-1).astype(jnp.int32), emb)
    wt_pad = jnp.pad(W.T, ((0, 0), (0, _VPAD - V)))
    b2_pad = jnp.pad(
        b.reshape(1, V), ((0, 0), (0, _VPAD - V)), constant_values=-1e30
    )
    return _tc_logsoftmax(s, wt_pad, b2_pad)
